# Initial kernel scaffold; baseline (speedup 1.0000x reference)
import jax
import jax.numpy as jnp
from jax.experimental import pallas as pl


def kernel(ptcloud):
    raise NotImplementedError("write your pallas kernel here")


# speedup vs baseline: 44.0292x; 44.0292x over previous
import functools

import jax
import jax.numpy as jnp
from jax import lax
from jax.experimental import pallas as pl
from jax.experimental.pallas import tpu as pltpu
from jax.experimental.pallas import tpu_sc as plsc

B = 32
N = 16384
GS = 64
NUM_CHUNKS = 4
SLABS_PER_CHUNK = GS // NUM_CHUNKS
CHUNK = SLABS_PER_CHUNK * GS * GS
L = 16
STEPS = N // L

_mesh = plsc.VectorSubcoreMesh(core_axis_name="c", subcore_axis_name="s")


@functools.partial(
    pl.kernel,
    mesh=_mesh,
    out_type=jax.ShapeDtypeStruct((B, GS * GS * GS), jnp.float32),
    scratch_types=[
        pltpu.VMEM((N,), jnp.float32),
        pltpu.VMEM((N,), jnp.float32),
        pltpu.VMEM((N,), jnp.float32),
        pltpu.VMEM((CHUNK,), jnp.float32),
    ],
    compiler_params=pltpu.CompilerParams(needs_layout_passes=False),
)
def _splat(xs_hbm, ys_hbm, zs_hbm, out_hbm, xs_v, ys_v, zs_v, grid_v):
    b = lax.axis_index("s") * 2 + lax.axis_index("c")
    pltpu.sync_copy(xs_hbm.at[b], xs_v)
    pltpu.sync_copy(ys_hbm.at[b], ys_v)
    pltpu.sync_copy(zs_hbm.at[b], zs_v)

    zeros = jnp.zeros((L,), jnp.float32)

    def zero_body(i, carry):
        grid_v[pl.ds(i * L, L)] = zeros
        return carry

    for k in range(NUM_CHUNKS):

        def step(i, carry, k=k):
            s = pl.ds(i * L, L)
            px = xs_v[s] * 32.0
            py = ys_v[s] * 32.0
            pz = zs_v[s] * 32.0
            tx = px.astype(jnp.int32)
            ty = py.astype(jnp.int32)
            tz = pz.astype(jnp.int32)
            flx = jnp.where(tx.astype(jnp.float32) > px, tx - 1, tx)
            fly = jnp.where(ty.astype(jnp.float32) > py, ty - 1, ty)
            flz = jnp.where(tz.astype(jnp.float32) > pz, tz - 1, tz)
            fx = px - flx.astype(jnp.float32)
            fy = py - fly.astype(jnp.float32)
            fz = pz - flz.astype(jnp.float32)
            ix = flx + 32
            iy = fly + 32
            iz = flz + 32
            wx0 = 1.0 - fx
            wy0 = 1.0 - fy
            wz0 = 1.0 - fz
            base = (ix << 12) + (iy << 6) + iz - (k << 16)
            ixp = ix + 1
            cx0 = (ix >> 4) == k
            cx1 = jnp.logical_and((ixp >> 4) == k, ixp <= GS - 1)
            my1 = iy <= GS - 2
            mz1 = iz <= GS - 2
            myz = jnp.logical_and(my1, mz1)
            wyz00 = wy0 * wz0
            wyz01 = wy0 * fz
            wyz10 = fy * wz0
            wyz11 = fy * fz
            corners = (
                (0, cx0, wx0 * wyz00),
                (1, jnp.logical_and(cx0, mz1), wx0 * wyz01),
                (GS, jnp.logical_and(cx0, my1), wx0 * wyz10),
                (GS + 1, jnp.logical_and(cx0, myz), wx0 * wyz11),
                (GS * GS, cx1, fx * wyz00),
                (GS * GS + 1, jnp.logical_and(cx1, mz1), fx * wyz01),
                (GS * GS + GS, jnp.logical_and(cx1, my1), fx * wyz10),
                (GS * GS + GS + 1, jnp.logical_and(cx1, myz), fx * wyz11),
            )
            for off, m, w in corners:
                idx = jnp.where(m, base + off, 0)
                plsc.addupdate_scatter(grid_v, [idx], w, mask=m)
            return carry

        lax.fori_loop(0, CHUNK // L, zero_body, 0)
        lax.fori_loop(0, STEPS, step, 0)
        pltpu.sync_copy(grid_v, out_hbm.at[b, pl.ds(k * CHUNK, CHUNK)])


def kernel(ptcloud):
    xs = ptcloud[:, :, 0]
    ys = ptcloud[:, :, 1]
    zs = ptcloud[:, :, 2]
    return _splat(xs, ys, zs)


# speedup vs baseline: 66.0710x; 1.5006x over previous
import functools

import jax
import jax.numpy as jnp
from jax import lax
from jax.experimental import pallas as pl
from jax.experimental.pallas import tpu as pltpu
from jax.experimental.pallas import tpu_sc as plsc

B = 32
N = 16384
GS = 64
NUM_CHUNKS = 4
SLABS_PER_CHUNK = GS // NUM_CHUNKS
CHUNK = SLABS_PER_CHUNK * GS * GS
L = 16
STEPS = N // L
UNROLL = 2
ZUNROLL = 16

_mesh = plsc.VectorSubcoreMesh(core_axis_name="c", subcore_axis_name="s")


@functools.partial(
    pl.kernel,
    mesh=_mesh,
    out_type=jax.ShapeDtypeStruct((B, GS * GS * GS), jnp.float32),
    scratch_types=[
        pltpu.VMEM((N,), jnp.float32),
        pltpu.VMEM((N,), jnp.float32),
        pltpu.VMEM((N,), jnp.float32),
        pltpu.VMEM((CHUNK,), jnp.float32),
    ],
    compiler_params=pltpu.CompilerParams(
        needs_layout_passes=False,
        disable_bounds_checks=True,
    ),
)
def _splat(xs_hbm, ys_hbm, zs_hbm, out_hbm, xs_v, ys_v, zs_v, grid_v):
    b = lax.axis_index("s") * 2 + lax.axis_index("c")
    pltpu.sync_copy(xs_hbm.at[b], xs_v)
    pltpu.sync_copy(ys_hbm.at[b], ys_v)
    pltpu.sync_copy(zs_hbm.at[b], zs_v)

    zeros = jnp.zeros((L,), jnp.float32)

    def zero_body(i, carry):
        b0 = i * (L * ZUNROLL)
        for j in range(ZUNROLL):
            grid_v[pl.ds(b0 + j * L, L)] = zeros
        return carry

    def one_step(i, k):
        s = pl.ds(i * L, L)
        px = xs_v[s] * 32.0
        py = ys_v[s] * 32.0
        pz = zs_v[s] * 32.0
        tx = px.astype(jnp.int32)
        ty = py.astype(jnp.int32)
        tz = pz.astype(jnp.int32)
        flx = jnp.where(tx.astype(jnp.float32) > px, tx - 1, tx)
        fly = jnp.where(ty.astype(jnp.float32) > py, ty - 1, ty)
        flz = jnp.where(tz.astype(jnp.float32) > pz, tz - 1, tz)
        fx = px - flx.astype(jnp.float32)
        fy = py - fly.astype(jnp.float32)
        fz = pz - flz.astype(jnp.float32)
        wx0 = 1.0 - fx
        wy0 = 1.0 - fy
        wz0 = 1.0 - fz
        cbias = 32 * GS * GS + 32 * GS + 32 - (k << 16)
        base = (flx << 12) + (fly << 6) + flz + cbias
        ix = flx + 32
        ixp = ix + 1
        cx0 = (ix >> 4) == k
        cx1 = jnp.logical_and((ixp >> 4) == k, ixp <= GS - 1)
        my1 = fly <= GS - 34
        mz1 = flz <= GS - 34
        myz = jnp.logical_and(my1, mz1)
        wyz00 = wy0 * wz0
        wyz01 = wy0 * fz
        wyz10 = fy * wz0
        wyz11 = fy * fz
        corners = (
            (0, cx0, wx0 * wyz00),
            (1, jnp.logical_and(cx0, mz1), wx0 * wyz01),
            (GS, jnp.logical_and(cx0, my1), wx0 * wyz10),
            (GS + 1, jnp.logical_and(cx0, myz), wx0 * wyz11),
            (GS * GS, cx1, fx * wyz00),
            (GS * GS + 1, jnp.logical_and(cx1, mz1), fx * wyz01),
            (GS * GS + GS, jnp.logical_and(cx1, my1), fx * wyz10),
            (GS * GS + GS + 1, jnp.logical_and(cx1, myz), fx * wyz11),
        )
        for off, m, w in corners:
            plsc.addupdate_scatter(grid_v, [base + off], w, mask=m)

    for k in range(NUM_CHUNKS):

        def step(i, carry, k=k):
            for j in range(UNROLL):
                one_step(i * UNROLL + j, k)
            return carry

        lax.fori_loop(0, CHUNK // (L * ZUNROLL), zero_body, 0)
        lax.fori_loop(0, STEPS // UNROLL, step, 0)
        pltpu.sync_copy(grid_v, out_hbm.at[b, pl.ds(k * CHUNK, CHUNK)])


def kernel(ptcloud):
    xs = ptcloud[:, :, 0]
    ys = ptcloud[:, :, 1]
    zs = ptcloud[:, :, 2]
    return _splat(xs, ys, zs)


# speedup vs baseline: 67.4729x; 1.0212x over previous
import functools

import jax
import jax.numpy as jnp
from jax import lax
from jax.experimental import pallas as pl
from jax.experimental.pallas import tpu as pltpu
from jax.experimental.pallas import tpu_sc as plsc

B = 32
N = 16384
GS = 64
NUM_CHUNKS = 4
SLABS_PER_CHUNK = GS // NUM_CHUNKS
CHUNK = SLABS_PER_CHUNK * GS * GS
L = 16
STEPS = N // L
UNROLL = 4
ZUNROLL = 16

_mesh = plsc.VectorSubcoreMesh(core_axis_name="c", subcore_axis_name="s")


@functools.partial(
    pl.kernel,
    mesh=_mesh,
    out_type=jax.ShapeDtypeStruct((B, GS * GS * GS), jnp.float32),
    scratch_types=[
        pltpu.VMEM((N,), jnp.float32),
        pltpu.VMEM((N,), jnp.float32),
        pltpu.VMEM((N,), jnp.float32),
        pltpu.VMEM((CHUNK,), jnp.float32),
    ],
    compiler_params=pltpu.CompilerParams(
        needs_layout_passes=False,
        disable_bounds_checks=True,
    ),
)
def _splat(xs_hbm, ys_hbm, zs_hbm, out_hbm, xs_v, ys_v, zs_v, grid_v):
    b = lax.axis_index("s") * 2 + lax.axis_index("c")
    pltpu.sync_copy(xs_hbm.at[b], xs_v)
    pltpu.sync_copy(ys_hbm.at[b], ys_v)
    pltpu.sync_copy(zs_hbm.at[b], zs_v)

    zeros = jnp.zeros((L,), jnp.float32)

    def zero_body(i, carry):
        b0 = i * (L * ZUNROLL)
        for j in range(ZUNROLL):
            grid_v[pl.ds(b0 + j * L, L)] = zeros
        return carry

    def one_step(i, k):
        s = pl.ds(i * L, L)
        px = xs_v[s] * 32.0
        py = ys_v[s] * 32.0
        pz = zs_v[s] * 32.0
        tx = px.astype(jnp.int32)
        ty = py.astype(jnp.int32)
        tz = pz.astype(jnp.int32)
        flx = jnp.where(tx.astype(jnp.float32) > px, tx - 1, tx)
        fly = jnp.where(ty.astype(jnp.float32) > py, ty - 1, ty)
        flz = jnp.where(tz.astype(jnp.float32) > pz, tz - 1, tz)
        fx = px - flx.astype(jnp.float32)
        fy = py - fly.astype(jnp.float32)
        fz = pz - flz.astype(jnp.float32)
        wx0 = 1.0 - fx
        wy0 = 1.0 - fy
        wz0 = 1.0 - fz
        cbias = 32 * GS * GS + 32 * GS + 32 - (k << 16)
        base = (flx << 12) + (fly << 6) + flz + cbias
        ix = flx + 32
        ixp = ix + 1
        cx0 = (ix >> 4) == k
        cx1 = jnp.logical_and((ixp >> 4) == k, ixp <= GS - 1)
        my1 = fly <= GS - 34
        mz1 = flz <= GS - 34
        myz = jnp.logical_and(my1, mz1)
        wyz00 = wy0 * wz0
        wyz01 = wy0 * fz
        wyz10 = fy * wz0
        wyz11 = fy * fz
        corners = (
            (0, cx0, wx0 * wyz00),
            (1, jnp.logical_and(cx0, mz1), wx0 * wyz01),
            (GS, jnp.logical_and(cx0, my1), wx0 * wyz10),
            (GS + 1, jnp.logical_and(cx0, myz), wx0 * wyz11),
            (GS * GS, cx1, fx * wyz00),
            (GS * GS + 1, jnp.logical_and(cx1, mz1), fx * wyz01),
            (GS * GS + GS, jnp.logical_and(cx1, my1), fx * wyz10),
            (GS * GS + GS + 1, jnp.logical_and(cx1, myz), fx * wyz11),
        )
        for off, m, w in corners:
            plsc.addupdate_scatter(grid_v, [base + off], w, mask=m)

    for k in range(NUM_CHUNKS):

        def step(i, carry, k=k):
            for j in range(UNROLL):
                one_step(i * UNROLL + j, k)
            return carry

        lax.fori_loop(0, CHUNK // (L * ZUNROLL), zero_body, 0)
        lax.fori_loop(0, STEPS // UNROLL, step, 0)
        pltpu.sync_copy(grid_v, out_hbm.at[b, pl.ds(k * CHUNK, CHUNK)])


def kernel(ptcloud):
    xs = ptcloud[:, :, 0]
    ys = ptcloud[:, :, 1]
    zs = ptcloud[:, :, 2]
    return _splat(xs, ys, zs)


# speedup vs baseline: 75.7456x; 1.1226x over previous
import functools

import jax
import jax.numpy as jnp
from jax import lax
from jax.experimental import pallas as pl
from jax.experimental.pallas import tpu as pltpu
from jax.experimental.pallas import tpu_sc as plsc

B = 32
N = 16384
GS = 64
CHUNK_LO = (0, 22, 43)
CHUNK_HI = (22, 43, 64)
MAX_CHUNK_W = 22
L = 16
HALF = N // 2
UNROLL = 4
ZUNROLL = 16

_mesh = plsc.VectorSubcoreMesh(core_axis_name="c", subcore_axis_name="s")


@functools.partial(
    pl.kernel,
    mesh=_mesh,
    out_type=jax.ShapeDtypeStruct((B, GS * GS * GS), jnp.float32),
    scratch_types=[
        pltpu.VMEM((HALF,), jnp.float32),
        pltpu.VMEM((HALF,), jnp.float32),
        pltpu.VMEM((HALF,), jnp.float32),
        pltpu.VMEM((MAX_CHUNK_W * GS * GS,), jnp.float32),
        pltpu.SemaphoreType.DMA,
    ],
    compiler_params=pltpu.CompilerParams(
        needs_layout_passes=False,
        disable_bounds_checks=True,
    ),
)
def _splat(xs_hbm, ys_hbm, zs_hbm, out_hbm, xs_v, ys_v, zs_v, grid_v, sem):
    b = lax.axis_index("s") * 2 + lax.axis_index("c")

    zeros = jnp.zeros((L,), jnp.float32)

    def zero_body(i, carry):
        b0 = i * (L * ZUNROLL)
        for j in range(ZUNROLL):
            grid_v[pl.ds(b0 + j * L, L)] = zeros
        return carry

    def one_step(i, k):
        lo, hi = CHUNK_LO[k], CHUNK_HI[k]
        s = pl.ds(i * L, L)
        px = xs_v[s] * 32.0
        py = ys_v[s] * 32.0
        pz = zs_v[s] * 32.0
        tx = px.astype(jnp.int32)
        ty = py.astype(jnp.int32)
        tz = pz.astype(jnp.int32)
        flx = jnp.where(tx.astype(jnp.float32) > px, tx - 1, tx)
        fly = jnp.where(ty.astype(jnp.float32) > py, ty - 1, ty)
        flz = jnp.where(tz.astype(jnp.float32) > pz, tz - 1, tz)
        fx = px - flx.astype(jnp.float32)
        fy = py - fly.astype(jnp.float32)
        fz = pz - flz.astype(jnp.float32)
        wx0 = 1.0 - fx
        wy0 = 1.0 - fy
        wz0 = 1.0 - fz
        cbias = 32 * GS * GS + 32 * GS + 32 - lo * GS * GS
        base = (flx << 12) + (fly << 6) + flz + cbias
        ix = flx + 32
        ixp = ix + 1
        if lo == 0:
            cx0 = ix < hi
            cx1 = ixp < hi
        elif hi == GS:
            cx0 = ix >= lo
            cx1 = jnp.logical_and(ixp >= lo, ixp <= GS - 1)
        else:
            cx0 = jnp.logical_and(ix >= lo, ix < hi)
            cx1 = jnp.logical_and(ixp >= lo, ixp < hi)
        my1 = fly <= GS - 34
        mz1 = flz <= GS - 34
        myz = jnp.logical_and(my1, mz1)
        wyz00 = wy0 * wz0
        wyz01 = wy0 * fz
        wyz10 = fy * wz0
        wyz11 = fy * fz
        corners = (
            (0, cx0, wx0 * wyz00),
            (1, jnp.logical_and(cx0, mz1), wx0 * wyz01),
            (GS, jnp.logical_and(cx0, my1), wx0 * wyz10),
            (GS + 1, jnp.logical_and(cx0, myz), wx0 * wyz11),
            (GS * GS, cx1, fx * wyz00),
            (GS * GS + 1, jnp.logical_and(cx1, mz1), fx * wyz01),
            (GS * GS + GS, jnp.logical_and(cx1, my1), fx * wyz10),
            (GS * GS + GS + 1, jnp.logical_and(cx1, myz), fx * wyz11),
        )
        for off, m, w in corners:
            plsc.addupdate_scatter(grid_v, [base + off], w, mask=m)

    out_off = 0
    for k in range(3):
        words = (CHUNK_HI[k] - CHUNK_LO[k]) * GS * GS
        lax.fori_loop(0, words // (L * ZUNROLL), zero_body, 0)
        for h in range(2):
            cx = pltpu.async_copy(xs_hbm.at[b, pl.ds(h * HALF, HALF)], xs_v, sem)
            cy = pltpu.async_copy(ys_hbm.at[b, pl.ds(h * HALF, HALF)], ys_v, sem)
            cz = pltpu.async_copy(zs_hbm.at[b, pl.ds(h * HALF, HALF)], zs_v, sem)
            cx.wait()
            cy.wait()
            cz.wait()

            def step(i, carry, k=k):
                for j in range(UNROLL):
                    one_step(i * UNROLL + j, k)
                return carry

            lax.fori_loop(0, HALF // (L * UNROLL), step, 0)
        pltpu.sync_copy(
            grid_v.at[pl.ds(0, words)], out_hbm.at[b, pl.ds(out_off, words)]
        )
        out_off += words


def kernel(ptcloud):
    xs = ptcloud[:, :, 0]
    ys = ptcloud[:, :, 1]
    zs = ptcloud[:, :, 2]
    return _splat(xs, ys, zs)
